# Initial kernel scaffold; baseline (speedup 1.0000x reference)
import jax
import jax.numpy as jnp
from jax.experimental import pallas as pl


def kernel(in_embed, out_embed, center_words, context_words, neg_samples):
    raise NotImplementedError("write your pallas kernel here")


# speedup vs baseline: 3.6938x; 3.6938x over previous
import functools

import jax
import jax.numpy as jnp
from jax import lax
from jax.experimental import pallas as pl
from jax.experimental.pallas import tpu as pltpu
from jax.experimental.pallas import tpu_sc as plsc

V = 100000
D = 300
DP = 304
B = 16384
K = 20
R = K + 1
RP = 24
NC = 2
NS = 16
NW = NC * NS
EPW = B // NW
SCH = 64
ECH = 4
NSUB = SCH // ECH
NSUP = EPW // SCH
NR = ECH * RP
LANES = 16
NCH = DP // LANES
RW = 32


def _permute(x, idx):
    dnums = lax.GatherDimensionNumbers(
        offset_dims=(), collapsed_slice_dims=(0,), start_index_map=(0,))
    return lax.gather(x, idx[:, None], dnums, slice_sizes=(1,),
                      mode=lax.GatherScatterMode.PROMISE_IN_BOUNDS)


def _sc_scores_kernel(in_hbm, out_hbm, cidx_hbm, ridx_hbm, scores_hbm,
                      cidx_v, ridx_v, scores_v, cbuf, rbuf):
    wid = lax.axis_index("s") * NC + lax.axis_index("c")
    base = wid * EPW
    pltpu.sync_copy(cidx_hbm.at[pl.ds(base, EPW)], cidx_v)
    pltpu.sync_copy(ridx_hbm.at[pl.ds(base * RP, EPW * RP)], ridx_v)

    lane = lax.iota(jnp.int32, LANES)

    def super_body(s, carry):
        pltpu.sync_copy(in_hbm.at[cidx_v.at[pl.ds(s * SCH, SCH)]], cbuf)

        def sub_body(c, carry2):
            pltpu.sync_copy(
                out_hbm.at[ridx_v.at[pl.ds(s * SCH * RP + c * NR, NR)]],
                rbuf)
            for j in range(ECH):
                crow = c * ECH + j
                cchunks = [cbuf[crow, pl.ds(o * LANES, LANES)]
                           for o in range(NCH)]
                packs = [jnp.zeros((LANES,), jnp.float32),
                         jnp.full((LANES,), jnp.inf, jnp.float32)]
                for k in range(R):
                    rrow = j * RP + k
                    acc = cchunks[0] * rbuf[rrow, pl.ds(0, LANES)]
                    for o in range(1, NCH):
                        acc = acc + cchunks[o] * rbuf[
                            rrow, pl.ds(o * LANES, LANES)]
                    for sft in (1, 2, 4, 8):
                        acc = acc + _permute(acc, lane ^ sft)
                    signed = acc if k == 0 else -acc
                    packs[k // LANES] = jnp.where(lane == (k % LANES),
                                                  signed, packs[k // LANES])
                e = s * SCH + c * ECH + j
                scores_v[e, pl.ds(0, LANES)] = packs[0]
                scores_v[e, pl.ds(LANES, LANES)] = packs[1]
            return carry2

        lax.fori_loop(0, NSUB, sub_body, 0)
        return carry

    lax.fori_loop(0, NSUP, super_body, 0)
    pltpu.sync_copy(scores_v, scores_hbm.at[pl.ds(base, EPW)])


def _make_sc_scores():
    mesh = plsc.VectorSubcoreMesh(core_axis_name="c", subcore_axis_name="s",
                                  num_cores=NC, num_subcores=NS)
    return functools.partial(
        pl.kernel,
        out_type=jax.ShapeDtypeStruct((B, RW), jnp.float32),
        mesh=mesh,
        compiler_params=pltpu.CompilerParams(use_tc_tiling_on_sc=False),
        scratch_types=[
            pltpu.VMEM((EPW,), jnp.int32),
            pltpu.VMEM((EPW * RP,), jnp.int32),
            pltpu.VMEM((EPW, RW), jnp.float32),
            pltpu.VMEM((SCH, DP), jnp.float32),
            pltpu.VMEM((NR, DP), jnp.float32),
        ],
    )(_sc_scores_kernel)


_sc_scores = _make_sc_scores()


_FLAT = B * RW
_ROWS = _FLAT // 128


def _loss_kernel(s_ref, o_ref):
    y = s_ref[:]
    ls = jnp.minimum(y, 0.0) - jnp.log1p(jnp.exp(-jnp.abs(y)))
    n = _ROWS
    while n > 8:
        n //= 2
        ls = ls[:n] + ls[n:2 * n]
    o_ref[0, 0] = -jnp.sum(ls) * (1.0 / B)


_loss = pl.pallas_call(
    _loss_kernel,
    out_shape=jax.ShapeDtypeStruct((1, 1), jnp.float32),
    out_specs=pl.BlockSpec(memory_space=pltpu.SMEM),
)


def kernel(in_embed, out_embed, center_words, context_words, neg_samples):
    in_p = jnp.pad(in_embed, ((0, 0), (0, DP - D)))
    out_p = jnp.pad(out_embed, ((0, 0), (0, DP - D)))
    cidx = center_words.astype(jnp.int32)
    ctx = context_words.astype(jnp.int32).reshape(B, 1)
    ridx = jnp.concatenate(
        [ctx, neg_samples.astype(jnp.int32), ctx, ctx, ctx],
        axis=1).reshape(B * RP)
    scores = _sc_scores(in_p, out_p, cidx, ridx)
    loss = _loss(scores.reshape(_ROWS, 128))
    return loss[0, 0]


# speedup vs baseline: 4.2488x; 1.1503x over previous
import functools

import jax
import jax.numpy as jnp
from jax import lax
from jax.experimental import pallas as pl
from jax.experimental.pallas import tpu as pltpu
from jax.experimental.pallas import tpu_sc as plsc

V = 100000
D = 300
DP = 304
B = 16384
K = 20
R = K + 1
RP = 24
NC = 2
NS = 16
NW = NC * NS
EPW = B // NW
SCH = 64
ECH = 2
NSUB = SCH // ECH
NSUP = EPW // SCH
NR = ECH * RP
LANES = 16
NCH = DP // LANES
RW = 32


def _permute(x, idx):
    dnums = lax.GatherDimensionNumbers(
        offset_dims=(), collapsed_slice_dims=(0,), start_index_map=(0,))
    return lax.gather(x, idx[:, None], dnums, slice_sizes=(1,),
                      mode=lax.GatherScatterMode.PROMISE_IN_BOUNDS)


def _sc_scores_kernel(in_hbm, out_hbm, cidx_hbm, ridx_hbm, scores_hbm,
                      cidx_v, ridx_v, scores_v, cbuf, rb0, rb1, sem0, sem1):
    wid = lax.axis_index("s") * NC + lax.axis_index("c")
    base = wid * EPW
    pltpu.sync_copy(cidx_hbm.at[pl.ds(base, EPW)], cidx_v)
    pltpu.sync_copy(ridx_hbm.at[pl.ds(base * RP, EPW * RP)], ridx_v)

    lane = lax.iota(jnp.int32, LANES)
    rbufs = [rb0, rb1]
    sems = [sem0, sem1]

    def super_body(s, carry):
        pltpu.sync_copy(in_hbm.at[cidx_v.at[pl.ds(s * SCH, SCH)]], cbuf)

        def issue(c, b):
            pltpu.async_copy(
                out_hbm.at[ridx_v.at[pl.ds(s * SCH * RP + c * NR, NR)]],
                rbufs[b], sems[b])

        def drain(c, b):
            pltpu.make_async_copy(
                out_hbm.at[ridx_v.at[pl.ds(s * SCH * RP + c * NR, NR)]],
                rbufs[b], sems[b]).wait()

        for b in range(2):
            issue(b, b)

        def sub_body(g, carry2):
            for b in range(2):
                c = g * 2 + b
                drain(c, b)
                rbuf = rbufs[b]
                _compute(s, c, rbuf)

                @pl.when(c + 2 < NSUB)
                def _():
                    issue(c + 2, b)
            return carry2

        def _compute(s, c, rbuf):
            for j in range(ECH):
                crow = c * ECH + j
                cchunks = [cbuf[crow, pl.ds(o * LANES, LANES)]
                           for o in range(NCH)]
                packs = [jnp.zeros((LANES,), jnp.float32),
                         jnp.full((LANES,), jnp.inf, jnp.float32)]
                for k in range(R):
                    rrow = j * RP + k
                    acc = cchunks[0] * rbuf[rrow, pl.ds(0, LANES)]
                    for o in range(1, NCH):
                        acc = acc + cchunks[o] * rbuf[
                            rrow, pl.ds(o * LANES, LANES)]
                    for sft in (1, 2, 4, 8):
                        acc = acc + _permute(acc, lane ^ sft)
                    signed = acc if k == 0 else -acc
                    packs[k // LANES] = jnp.where(lane == (k % LANES),
                                                  signed, packs[k // LANES])
                e = s * SCH + c * ECH + j
                scores_v[e, pl.ds(0, LANES)] = packs[0]
                scores_v[e, pl.ds(LANES, LANES)] = packs[1]

        lax.fori_loop(0, NSUB // 2, sub_body, 0)
        return carry

    lax.fori_loop(0, NSUP, super_body, 0)
    pltpu.sync_copy(scores_v, scores_hbm.at[pl.ds(base, EPW)])


def _make_sc_scores():
    mesh = plsc.VectorSubcoreMesh(core_axis_name="c", subcore_axis_name="s",
                                  num_cores=NC, num_subcores=NS)
    return functools.partial(
        pl.kernel,
        out_type=jax.ShapeDtypeStruct((B, RW), jnp.float32),
        mesh=mesh,
        compiler_params=pltpu.CompilerParams(use_tc_tiling_on_sc=False),
        scratch_types=[
            pltpu.VMEM((EPW,), jnp.int32),
            pltpu.VMEM((EPW * RP,), jnp.int32),
            pltpu.VMEM((EPW, RW), jnp.float32),
            pltpu.VMEM((SCH, DP), jnp.float32),
            pltpu.VMEM((NR, DP), jnp.float32),
            pltpu.VMEM((NR, DP), jnp.float32),
            pltpu.SemaphoreType.DMA,
            pltpu.SemaphoreType.DMA,
        ],
    )(_sc_scores_kernel)


_sc_scores = _make_sc_scores()


_FLAT = B * RW
_ROWS = _FLAT // 128


def _loss_kernel(s_ref, o_ref):
    y = s_ref[:]
    ls = jnp.minimum(y, 0.0) - jnp.log1p(jnp.exp(-jnp.abs(y)))
    n = _ROWS
    while n > 8:
        n //= 2
        ls = ls[:n] + ls[n:2 * n]
    o_ref[0, 0] = -jnp.sum(ls) * (1.0 / B)


_loss = pl.pallas_call(
    _loss_kernel,
    out_shape=jax.ShapeDtypeStruct((1, 1), jnp.float32),
    out_specs=pl.BlockSpec(memory_space=pltpu.SMEM),
)


def kernel(in_embed, out_embed, center_words, context_words, neg_samples):
    in_p = jnp.pad(in_embed, ((0, 0), (0, DP - D)))
    out_p = jnp.pad(out_embed, ((0, 0), (0, DP - D)))
    cidx = center_words.astype(jnp.int32)
    ctx = context_words.astype(jnp.int32).reshape(B, 1)
    ridx = jnp.concatenate(
        [ctx, neg_samples.astype(jnp.int32), ctx, ctx, ctx],
        axis=1).reshape(B * RP)
    scores = _sc_scores(in_p, out_p, cidx, ridx)
    loss = _loss(scores.reshape(_ROWS, 128))
    return loss[0, 0]


# speedup vs baseline: 4.2666x; 1.0042x over previous
import functools

import jax
import jax.numpy as jnp
from jax import lax
from jax.experimental import pallas as pl
from jax.experimental.pallas import tpu as pltpu
from jax.experimental.pallas import tpu_sc as plsc

V = 100000
D = 300
DP = 304
B = 16384
K = 20
R = K + 1
RP = 24
NC = 2
NS = 16
NW = NC * NS
EPW = B // NW
SCH = 64
ECH = 2
NSUB = SCH // ECH
NSUP = EPW // SCH
NR = ECH * RP
LANES = 16
NCH = DP // LANES
RW = 32


def _permute(x, idx):
    dnums = lax.GatherDimensionNumbers(
        offset_dims=(), collapsed_slice_dims=(0,), start_index_map=(0,))
    return lax.gather(x, idx[:, None], dnums, slice_sizes=(1,),
                      mode=lax.GatherScatterMode.PROMISE_IN_BOUNDS)


def _sc_scores_kernel(in_hbm, out_hbm, cidx_hbm, ridx_hbm, scores_hbm,
                      cidx_v, ridx_v, scores_v, cbuf, rb0, rb1, sem0, sem1):
    wid = lax.axis_index("s") * NC + lax.axis_index("c")
    base = wid * EPW
    pltpu.sync_copy(cidx_hbm.at[pl.ds(base, EPW)], cidx_v)
    pltpu.sync_copy(ridx_hbm.at[pl.ds(base * RP, EPW * RP)], ridx_v)

    lane = lax.iota(jnp.int32, LANES)
    rbufs = [rb0, rb1]
    sems = [sem0, sem1]

    def super_body(s, carry):
        pltpu.sync_copy(in_hbm.at[cidx_v.at[pl.ds(s * SCH, SCH)]], cbuf)

        def issue(c, b):
            pltpu.async_copy(
                out_hbm.at[ridx_v.at[pl.ds(s * SCH * RP + c * NR, NR)]],
                rbufs[b], sems[b])

        def drain(c, b):
            pltpu.make_async_copy(
                out_hbm.at[ridx_v.at[pl.ds(s * SCH * RP + c * NR, NR)]],
                rbufs[b], sems[b]).wait()

        for b in range(2):
            issue(b, b)

        def sub_body(g, carry2):
            for b in range(2):
                c = g * 2 + b
                drain(c, b)
                rbuf = rbufs[b]
                _compute(s, c, rbuf)

                @pl.when(c + 2 < NSUB)
                def _():
                    issue(c + 2, b)
            return carry2

        def _compute(s, c, rbuf):
            for j in range(ECH):
                crow = c * ECH + j
                cchunks = [cbuf[crow, pl.ds(o * LANES, LANES)]
                           for o in range(NCH)]
                packs = [jnp.zeros((LANES,), jnp.float32),
                         jnp.full((LANES,), jnp.inf, jnp.float32)]
                for k in range(R):
                    rrow = j * RP + k
                    acc = cchunks[0] * rbuf[rrow, pl.ds(0, LANES)]
                    for o in range(1, NCH):
                        acc = acc + cchunks[o] * rbuf[
                            rrow, pl.ds(o * LANES, LANES)]
                    for sft in (1, 2, 4, 8):
                        acc = acc + _permute(acc, lane ^ sft)
                    signed = acc if k == 0 else -acc
                    packs[k // LANES] = jnp.where(lane == (k % LANES),
                                                  signed, packs[k // LANES])
                e = s * SCH + c * ECH + j
                scores_v[e, pl.ds(0, LANES)] = packs[0]
                scores_v[e, pl.ds(LANES, LANES)] = packs[1]

        lax.fori_loop(0, NSUB // 2, sub_body, 0)
        return carry

    lax.fori_loop(0, NSUP, super_body, 0)
    pltpu.sync_copy(scores_v, scores_hbm.at[pl.ds(base, EPW)])


def _make_sc_scores():
    mesh = plsc.VectorSubcoreMesh(core_axis_name="c", subcore_axis_name="s",
                                  num_cores=NC, num_subcores=NS)
    return functools.partial(
        pl.kernel,
        out_type=jax.ShapeDtypeStruct((B, RW), jnp.float32),
        mesh=mesh,
        compiler_params=pltpu.CompilerParams(use_tc_tiling_on_sc=False),
        scratch_types=[
            pltpu.VMEM((EPW,), jnp.int32),
            pltpu.VMEM((EPW * RP,), jnp.int32),
            pltpu.VMEM((EPW, RW), jnp.float32),
            pltpu.VMEM((SCH, DP), jnp.float32),
            pltpu.VMEM((NR, DP), jnp.float32),
            pltpu.VMEM((NR, DP), jnp.float32),
            pltpu.SemaphoreType.DMA,
            pltpu.SemaphoreType.DMA,
        ],
    )(_sc_scores_kernel)


_sc_scores = _make_sc_scores()


_FLAT = B * RW
_ROWS = _FLAT // 128


def _loss_kernel(s_ref, o_ref):
    y = s_ref[:]
    ls = jnp.minimum(y, 0.0) - jnp.log1p(jnp.exp(-jnp.abs(y)))
    n = _ROWS
    while n > 8:
        n //= 2
        ls = ls[:n] + ls[n:2 * n]
    o_ref[0, 0] = -jnp.sum(ls) * (1.0 / B)


_loss = pl.pallas_call(
    _loss_kernel,
    out_shape=jax.ShapeDtypeStruct((1, 1), jnp.float32),
    out_specs=pl.BlockSpec(memory_space=pltpu.SMEM),
)


def kernel(in_embed, out_embed, center_words, context_words, neg_samples):
    in_p = jnp.pad(in_embed, ((0, 0), (0, DP - D)))
    out_p = jnp.pad(out_embed, ((0, 0), (0, DP - D)))
    cidx = center_words.astype(jnp.int32)
    ctx = context_words.astype(jnp.int32).reshape(B, 1)
    pads = jnp.concatenate([(ctx + 1) % V, (ctx + 2) % V, (ctx + 3) % V],
                           axis=1)
    ridx = jnp.concatenate(
        [ctx, neg_samples.astype(jnp.int32), pads],
        axis=1).reshape(B * RP)
    scores = _sc_scores(in_p, out_p, cidx, ridx)
    loss = _loss(scores.reshape(_ROWS, 128))
    return loss[0, 0]
